# baseline (speedup 1.0000x reference)
import functools

import numpy as np
import jax
import jax.numpy as jnp
from jax import lax
from jax.experimental import pallas as pl
from jax.experimental.pallas import tpu as pltpu
from jax.experimental.pallas import tpu_sc as plsc

B = 64
C = 80
CPAD = 128
D = 128
P = B * (B - 1) // 2
PPAD = 2048

_OFF = [i * (B - 1) - i * (i - 1) // 2 for i in range(B)]


def _gram_body(x_ref, out_ref):
    x = x_ref[...]
    g = lax.dot_general(
        x, x, (((2,), (2,)), ((1,), (1,))),
        preferred_element_type=jnp.float32)
    nsq = jnp.sum(x * x, axis=2).T
    den = nsq[:, :, None] * nsq[:, None, :]
    gn = g * lax.rsqrt(jnp.maximum(den, 1e-18))
    t = gn.reshape(C, B * B).T
    out_ref[:, :C] = t


def _gram_tc(x):
    return pl.pallas_call(
        _gram_body,
        out_shape=jax.ShapeDtypeStruct((B * B, CPAD), jnp.float32),
    )(x)


_NC = 2
_NS = 16
_NW = _NC * _NS
_BPW = PPAD // _NW

_mesh = plsc.VectorSubcoreMesh(core_axis_name="c", subcore_axis_name="s")


@functools.partial(
    pl.kernel,
    mesh=_mesh,
    out_type=jax.ShapeDtypeStruct((P * CPAD,), jnp.float32),
    scratch_types=[
        pltpu.VMEM(((B - 1) * CPAD,), jnp.float32),
        pltpu.SemaphoreType.DMA,
        pltpu.SemaphoreType.DMA,
        pltpu.SemaphoreType.DMA,
        pltpu.SemaphoreType.DMA,
    ],
)
def _pair_gather_sc(table_hbm, out_hbm, buf_v, sem_a, sem_b, sem_c, sem_d):
    wid = lax.axis_index("s") * _NC + lax.axis_index("c")

    def seg(i):
        n = (B - 1) - i
        return (i * (B + 1) + 1) * CPAD, _OFF[i] * CPAD, n * CPAD

    for w in range(_NW):
        ia, ib = w, (B - 1) - w
        sa, da, na = seg(ia)
        sb, db, nb = seg(ib)

        @pl.when(wid == w)
        def _(sa=sa, da=da, na=na, sb=sb, db=db, nb=nb):
            ca = pltpu.make_async_copy(
                table_hbm.at[pl.ds(sa, na)], buf_v.at[pl.ds(0, na)], sem_a)
            ca.start()
            if nb:
                cb = pltpu.make_async_copy(
                    table_hbm.at[pl.ds(sb, nb)], buf_v.at[pl.ds(na, nb)],
                    sem_b)
                cb.start()
            ca.wait()
            sta = pltpu.make_async_copy(
                buf_v.at[pl.ds(0, na)], out_hbm.at[pl.ds(da, na)], sem_c)
            sta.start()
            if nb:
                cb.wait()
                stb = pltpu.make_async_copy(
                    buf_v.at[pl.ds(na, nb)], out_hbm.at[pl.ds(db, nb)],
                    sem_d)
                stb.start()
                stb.wait()
            sta.wait()


def _slice_body(x_ref, out_ref):
    out_ref[...] = x_ref[:, :C]


def _slice_tc(x):
    return pl.pallas_call(
        _slice_body,
        out_shape=jax.ShapeDtypeStruct((P, C), jnp.float32),
    )(x)


def kernel(input, target):
    table = _gram_tc(input)
    out = _pair_gather_sc(table.reshape(B * B * CPAD))
    return _slice_tc(out.reshape(P, CPAD))


# speedup vs baseline: 1.0759x; 1.0759x over previous
import functools

import numpy as np
import jax
import jax.numpy as jnp
from jax import lax
from jax.experimental import pallas as pl
from jax.experimental.pallas import tpu as pltpu
from jax.experimental.pallas import tpu_sc as plsc

B = 64
C = 80
CPAD = 128
D = 128
P = B * (B - 1) // 2
PPAD = 2048

_OFF = [i * (B - 1) - i * (i - 1) // 2 for i in range(B)]


def _gram_body(x_ref, out_ref):
    x = x_ref[...]
    g = lax.dot_general(
        x, x, (((2,), (2,)), ((1,), (1,))),
        preferred_element_type=jnp.float32)
    nsq = jnp.sum(x * x, axis=2).T
    den = nsq[:, :, None] * nsq[:, None, :]
    gn = g * lax.rsqrt(jnp.maximum(den, 1e-18))
    t = gn.reshape(C, B * B).T
    out_ref[:, :C] = t


def _gram_tc(x):
    return pl.pallas_call(
        _gram_body,
        out_shape=jax.ShapeDtypeStruct((B * B, CPAD), jnp.float32),
    )(x)


_NC = 2
_NS = 16
_NW = _NC * _NS
_BPW = PPAD // _NW

_mesh = plsc.VectorSubcoreMesh(core_axis_name="c", subcore_axis_name="s")


@functools.partial(
    pl.kernel,
    mesh=_mesh,
    out_type=jax.ShapeDtypeStruct((P * CPAD,), jnp.float32),
    scratch_types=[
        pltpu.VMEM(((B - 1) * CPAD,), jnp.float32),
        pltpu.SemaphoreType.DMA,
        pltpu.SemaphoreType.DMA,
        pltpu.SemaphoreType.DMA,
        pltpu.SemaphoreType.DMA,
    ],
)
def _pair_gather_sc(table_hbm, out_hbm, buf_v, sem_a, sem_b, sem_c, sem_d):
    wid = lax.axis_index("s") * _NC + lax.axis_index("c")

    def seg(i):
        n = (B - 1) - i
        return (i * (B + 1) + 1) * CPAD, _OFF[i] * CPAD, n * CPAD

    for w in range(_NW):
        ia, ib = w, (B - 1) - w
        sa, da, na = seg(ia)
        sb, db, nb = seg(ib)

        @pl.when(wid == w)
        def _(sa=sa, da=da, na=na, sb=sb, db=db, nb=nb):
            ca = pltpu.make_async_copy(
                table_hbm.at[pl.ds(sa, na)], buf_v.at[pl.ds(0, na)], sem_a)
            ca.start()
            if nb:
                cb = pltpu.make_async_copy(
                    table_hbm.at[pl.ds(sb, nb)], buf_v.at[pl.ds(na, nb)],
                    sem_b)
                cb.start()
            ca.wait()
            sta = pltpu.make_async_copy(
                buf_v.at[pl.ds(0, na)], out_hbm.at[pl.ds(da, na)], sem_c)
            sta.start()
            if nb:
                cb.wait()
                stb = pltpu.make_async_copy(
                    buf_v.at[pl.ds(na, nb)], out_hbm.at[pl.ds(db, nb)],
                    sem_d)
                stb.start()
                stb.wait()
            sta.wait()


def kernel(input, target):
    table = _gram_tc(input)
    out = _pair_gather_sc(table.reshape(B * B * CPAD))
    return out.reshape(P, CPAD)[:, :C]
